# baseline (speedup 1.0000x reference)
import functools

import jax
import jax.numpy as jnp
from jax import lax
from jax.experimental import pallas as pl
from jax.experimental.pallas import tpu as pltpu
from jax.experimental.pallas import tpu_sc as plsc

N = 10000
E = 100000
IN_C = 32
OUT_C = 32
ATTR = 16
H1 = 256
H2 = 1024

NC = 2
NS = 16
NW = NC * NS
PK = 128 // IN_C

G_CHUNK = 128
G_PAD = 102400
G_BPW = G_PAD // NW
G_NCH = G_BPW // G_CHUNK

T_EDGE = 2048
TC_GRID = -(-E // T_EDGE)
E_MSG = TC_GRID * T_EDGE

S_CHUNK = 112
S_NCH = 28
S_BPW = S_CHUNK * S_NCH


def _gather_body(x_hbm, idx_hbm, out_hbm, idx_v, rows_v, sem):
    wid = lax.axis_index("s") * NC + lax.axis_index("c")
    pltpu.sync_copy(idx_hbm.at[pl.ds(wid * G_BPW, G_BPW)], idx_v)

    copies = [
        pltpu.async_copy(x_hbm.at[idx_v.at[pl.ds(j * G_CHUNK, G_CHUNK)]],
                         rows_v.at[pl.ds(j * G_CHUNK, G_CHUNK)], sem)
        for j in range(G_NCH)
    ]
    for c in copies:
        c.wait()
    pltpu.sync_copy(rows_v, out_hbm.at[pl.ds(wid * G_BPW, G_BPW)])


def _sc_gather(x, idx):
    mesh = plsc.VectorSubcoreMesh(core_axis_name="c", subcore_axis_name="s")
    k = functools.partial(
        pl.kernel, mesh=mesh,
        out_type=jax.ShapeDtypeStruct((G_PAD, IN_C), jnp.float32),
        scratch_types=[
            pltpu.VMEM((G_BPW,), jnp.int32),
            pltpu.VMEM((G_BPW, IN_C), jnp.float32),
            pltpu.SemaphoreType.DMA,
        ],
        compiler_params=pltpu.CompilerParams(use_tc_tiling_on_sc=False),
    )(_gather_body)
    return k(x, idx)


def _scatter_body(msg_hbm, dst_hbm, zeros_hbm, out_hbm,
                  idx_v, msg_v, acc, sem):
    cid = lax.axis_index("c")
    sid = lax.axis_index("s")
    wid = sid * NC + cid
    pltpu.sync_copy(dst_hbm.at[wid], idx_v)
    pltpu.sync_copy(msg_hbm.at[pl.ds(wid * S_BPW, S_BPW)], msg_v)

    @pl.when(sid < 10)
    def _():
        pltpu.sync_copy(zeros_hbm.at[pl.ds(0, 1000)],
                        acc.at[pl.ds(sid * 1000, 1000)])

    @pl.when(sid == 10)
    def _():
        pltpu.sync_copy(zeros_hbm.at[pl.ds(0, 8)],
                        acc.at[pl.ds(N, 8)])

    plsc.subcore_barrier()

    def body(j, carry):
        pltpu.sync_copy(msg_v.at[pl.ds(j * S_CHUNK, S_CHUNK)],
                        acc.at[idx_v.at[j]], add=True)
        return carry

    lax.fori_loop(0, S_NCH, body, 0)
    plsc.subcore_barrier()

    @pl.when(sid == 0)
    def _():
        pltpu.sync_copy(acc.at[pl.ds(0, N)], out_hbm.at[cid])


def _sc_scatter(msg, dst3, zeros):
    mesh = plsc.VectorSubcoreMesh(core_axis_name="c", subcore_axis_name="s")
    k = functools.partial(
        pl.kernel, mesh=mesh,
        out_type=jax.ShapeDtypeStruct((NC, N, OUT_C), jnp.float32),
        scratch_types=[
            pltpu.VMEM((S_NCH, S_CHUNK), jnp.int32),
            pltpu.VMEM((S_BPW, OUT_C), jnp.float32),
            pltpu.VMEM_SHARED((N + 8, OUT_C), jnp.float32),
            pltpu.SemaphoreType.DMA,
        ],
        compiler_params=pltpu.CompilerParams(use_tc_tiling_on_sc=False),
    )(_scatter_body)
    return k(msg, dst3, zeros)


def _elu(v):
    return jnp.maximum(v, jnp.exp(jnp.minimum(v, 0.0)) - 1.0)


_DN_T = (((0,), (0,)), ((), ()))


def _mlp_body(attrT_ref, xj_ref, w1_ref, b1_ref, w2_ref, b2_ref,
              w3_ref, b3_ref, q_ref, p_ref, out_ref):
    h = _elu(lax.dot_general(attrT_ref[...], w1_ref[...], _DN_T,
                             preferred_element_type=jnp.float32) + b1_ref[...])
    h = _elu(jnp.dot(h, w2_ref[...],
                     preferred_element_type=jnp.float32) + b2_ref[...])
    w = _elu(jnp.dot(h, w3_ref[...],
                     preferred_element_type=jnp.float32) + b3_ref[...])
    xb = jnp.dot(xj_ref[...], q_ref[...], preferred_element_type=jnp.float32)
    out_ref[...] = jnp.dot(xb * w, p_ref[...],
                           preferred_element_type=jnp.float32)


def _tc_mlp_msg(attrT, xj, W1, b1, W2, b2, W3, b3, Q, P):
    whole = lambda shape: pl.BlockSpec(shape, lambda g: (0, 0))
    return pl.pallas_call(
        _mlp_body,
        grid=(TC_GRID,),
        in_specs=[
            pl.BlockSpec((ATTR, T_EDGE), lambda g: (0, g)),
            pl.BlockSpec((T_EDGE, IN_C), lambda g: (g, 0)),
            whole((ATTR, H1)), whole((1, H1)),
            whole((H1, H2)), whole((1, H2)),
            whole((H2, IN_C * OUT_C)),
            whole((1, IN_C * OUT_C)),
            whole((IN_C, IN_C * OUT_C)), whole((IN_C * OUT_C, OUT_C)),
        ],
        out_specs=pl.BlockSpec((T_EDGE, OUT_C), lambda g: (g, 0)),
        out_shape=jax.ShapeDtypeStruct((E_MSG, OUT_C), jnp.float32),
    )(attrT, xj, W1, b1, W2, b2, W3, b3, Q, P)


def _root_body(x_ref, root_ref, bias_ref, out_ref):
    out_ref[...] = jnp.dot(x_ref[...], root_ref[...],
                           preferred_element_type=jnp.float32) + bias_ref[...]


def _tc_root(x, root, bias_r):
    return pl.pallas_call(
        _root_body,
        out_shape=jax.ShapeDtypeStruct((N, OUT_C), jnp.float32),
    )(x, root, bias_r)


def kernel(x, edge_index, edge_attr, W1, b1, W2, b2, W3, b3, root, bias):
    src = edge_index[0]
    dst = edge_index[1]
    src_p = jnp.pad(src, (0, G_PAD - E))
    dst3 = jnp.pad(dst, (0, E_MSG - E),
                   constant_values=N).reshape(NW, S_NCH, S_CHUNK)

    Q = jnp.kron(jnp.eye(IN_C, dtype=jnp.float32),
                 jnp.ones((1, OUT_C), dtype=jnp.float32))
    P = jnp.kron(jnp.ones((IN_C, 1), dtype=jnp.float32),
                 jnp.eye(OUT_C, dtype=jnp.float32))

    xj = _sc_gather(x, src_p)
    msg = _tc_mlp_msg(edge_attr.T, xj, W1, b1.reshape(1, H1),
                      W2, b2.reshape(1, H2), W3,
                      b3.reshape(1, IN_C * OUT_C), Q, P)
    out0 = _tc_root(x, root, bias.reshape(1, OUT_C))
    zeros = jnp.zeros((1000, OUT_C), dtype=jnp.float32)
    partials = _sc_scatter(msg, dst3, zeros)
    return partials[0] + partials[1] + out0
